# baseline (speedup 1.0000x reference)
import functools

import jax
import jax.numpy as jnp
from jax import lax
from jax.experimental import pallas as pl
from jax.experimental.pallas import tpu as pltpu
from jax.experimental.pallas import tpu_sc as plsc

B, H, S, D, Q = 16, 8, 2048, 128, 1
BH = B * H
CHUNK = 256
NC, NS = 2, 16
PAIRS_PER_TILE = BH // (NC * NS)
PAIRS_PER_SC = BH // NC
PATCH_TILES = PAIRS_PER_SC // 16


def _body(k_hbm, v_hbm, lk_hbm, lv_hbm, pos_hbm, ok_hbm, ov_hbm,
          buf0, buf1, rows_v, pos_v, isem0, isem1, osem0, osem1, psem):
    c = lax.axis_index("c")
    s = lax.axis_index("s")

    bufs = (buf0, buf1)
    isems = (isem0, isem1)
    osems = (osem0, osem1)

    chunk_list = []
    for i in range(PAIRS_PER_TILE):
        pair = c * PAIRS_PER_SC + s * PAIRS_PER_TILE + i
        base = pair * S
        for j in range(S // CHUNK):
            row0 = base + j * CHUNK
            chunk_list.append((k_hbm, ok_hbm, row0))
            chunk_list.append((v_hbm, ov_hbm, row0))

    pending = [None, None]
    for t, (src, dst, row0) in enumerate(chunk_list):
        kbuf = t % 2
        if pending[kbuf] is not None:
            pending[kbuf].wait()
        cp_in = pltpu.make_async_copy(
            src.at[pl.ds(row0, CHUNK)], bufs[kbuf], isems[kbuf])
        cp_in.start()
        cp_in.wait()
        cp_out = pltpu.make_async_copy(
            bufs[kbuf], dst.at[pl.ds(row0, CHUNK)], osems[kbuf])
        cp_out.start()
        pending[kbuf] = cp_out
    for p in pending:
        if p is not None:
            p.wait()

    plsc.subcore_barrier()

    pltpu.sync_copy(pos_hbm, pos_v)
    iota = lax.iota(jnp.int32, 16)

    def patch(latest_hbm, out_hbm, tile):
        pair_vec = c * PAIRS_PER_SC + tile * 16 + iota
        b_vec = lax.shift_right_logical(pair_vec, 3)
        pos_vals = plsc.load_gather(pos_v, [b_vec])
        dst_vec = pair_vec * S + pos_vals
        cp_g = pltpu.make_async_copy(latest_hbm.at[pair_vec], rows_v, psem)
        cp_g.start()
        cp_g.wait()
        cp_s = pltpu.make_async_copy(rows_v, out_hbm.at[dst_vec], psem)
        cp_s.start()
        cp_s.wait()

    for tile in range(PATCH_TILES):
        @pl.when(s == tile)
        def _(tile=tile):
            patch(lk_hbm, ok_hbm, tile)

        @pl.when(s == PATCH_TILES + tile)
        def _(tile=tile):
            patch(lv_hbm, ov_hbm, tile)


def kernel(k_cache, v_cache, latest_k, latest_v, position_ids):
    pos = position_ids.reshape(B).astype(jnp.int32)
    k2 = k_cache.reshape(BH * S, D)
    v2 = v_cache.reshape(BH * S, D)
    lk2 = latest_k.reshape(BH, D)
    lv2 = latest_v.reshape(BH, D)
    mesh = plsc.VectorSubcoreMesh(core_axis_name="c", subcore_axis_name="s")
    run = pl.kernel(
        _body,
        out_type=[
            jax.ShapeDtypeStruct((BH * S, D), k_cache.dtype),
            jax.ShapeDtypeStruct((BH * S, D), v_cache.dtype),
        ],
        mesh=mesh,
        compiler_params=pltpu.CompilerParams(needs_layout_passes=False),
        scratch_types=[
            pltpu.VMEM((CHUNK, D), jnp.float32),
            pltpu.VMEM((CHUNK, D), jnp.float32),
            pltpu.VMEM((16, D), jnp.float32),
            pltpu.VMEM((16,), jnp.int32),
            pltpu.SemaphoreType.DMA,
            pltpu.SemaphoreType.DMA,
            pltpu.SemaphoreType.DMA,
            pltpu.SemaphoreType.DMA,
            pltpu.SemaphoreType.DMA,
        ],
    )
    k_new, v_new = run(k2, v2, lk2, lv2, pos)
    return (k_new.reshape(B, H, S, D), v_new.reshape(B, H, S, D))


# speedup vs baseline: 1.0992x; 1.0992x over previous
import jax
import jax.numpy as jnp
from jax import lax
from jax.experimental import pallas as pl
from jax.experimental.pallas import tpu as pltpu
from jax.experimental.pallas import tpu_sc as plsc

B, H, S, D, Q = 16, 8, 2048, 128, 1
BH = B * H
BS = 1024
CHUNK = 256
NC, NS = 2, 16
PAIRS_PER_TILE = BH // (NC * NS)
PAIRS_PER_SC = BH // NC
PATCH_TILES = PAIRS_PER_SC // 16


def _tc_body(pos_ref, k_ref, lk_ref, ok_ref):
    b = pl.program_id(0)
    s = pl.program_id(1)
    ok_ref[...] = k_ref[...]
    local = pos_ref[b] - s * BS

    @pl.when((local >= 0) & (local < BS))
    def _():
        ok_ref[0, :, pl.ds(local, 1), :] = lk_ref[0]


def _tc_update(cache, latest, pos):
    grid_spec = pltpu.PrefetchScalarGridSpec(
        num_scalar_prefetch=1,
        grid=(B, S // BS),
        in_specs=[
            pl.BlockSpec((1, H, BS, D), lambda b, s, p: (b, 0, s, 0)),
            pl.BlockSpec((1, H, Q, D), lambda b, s, p: (b, 0, 0, 0)),
        ],
        out_specs=pl.BlockSpec((1, H, BS, D), lambda b, s, p: (b, 0, s, 0)),
    )
    return pl.pallas_call(
        _tc_body,
        grid_spec=grid_spec,
        out_shape=jax.ShapeDtypeStruct((B, H, S, D), cache.dtype),
    )(pos, cache, latest)


def _sc_body(v_hbm, lv_hbm, pos_hbm, ov_hbm,
             buf0, buf1, rows_v, pos_v, isem0, isem1, osem0, osem1, psem):
    c = lax.axis_index("c")
    s = lax.axis_index("s")

    bufs = (buf0, buf1)
    isems = (isem0, isem1)
    osems = (osem0, osem1)

    chunk_rows = []
    for i in range(PAIRS_PER_TILE):
        pair = c * PAIRS_PER_SC + s * PAIRS_PER_TILE + i
        for j in range(S // CHUNK):
            chunk_rows.append(pair * S + j * CHUNK)

    pending = [None, None]
    for t, row0 in enumerate(chunk_rows):
        kbuf = t % 2
        if pending[kbuf] is not None:
            pending[kbuf].wait()
        cp_in = pltpu.make_async_copy(
            v_hbm.at[pl.ds(row0, CHUNK)], bufs[kbuf], isems[kbuf])
        cp_in.start()
        cp_in.wait()
        cp_out = pltpu.make_async_copy(
            bufs[kbuf], ov_hbm.at[pl.ds(row0, CHUNK)], osems[kbuf])
        cp_out.start()
        pending[kbuf] = cp_out
    for p in pending:
        if p is not None:
            p.wait()

    plsc.subcore_barrier()

    pltpu.sync_copy(pos_hbm, pos_v)
    iota = lax.iota(jnp.int32, 16)

    for tile in range(PATCH_TILES):
        @pl.when(s == tile)
        def _(tile=tile):
            pair_vec = c * PAIRS_PER_SC + tile * 16 + iota
            b_vec = lax.shift_right_logical(pair_vec, 3)
            pos_vals = plsc.load_gather(pos_v, [b_vec])
            dst_vec = pair_vec * S + pos_vals
            cp_g = pltpu.make_async_copy(lv_hbm.at[pair_vec], rows_v, psem)
            cp_g.start()
            cp_g.wait()
            cp_s = pltpu.make_async_copy(rows_v, ov_hbm.at[dst_vec], psem)
            cp_s.start()
            cp_s.wait()


def _sc_update(cache2, latest2, pos):
    mesh = plsc.VectorSubcoreMesh(core_axis_name="c", subcore_axis_name="s")
    run = pl.kernel(
        _sc_body,
        out_type=jax.ShapeDtypeStruct((BH * S, D), cache2.dtype),
        mesh=mesh,
        compiler_params=pltpu.CompilerParams(needs_layout_passes=False),
        scratch_types=[
            pltpu.VMEM((CHUNK, D), jnp.float32),
            pltpu.VMEM((CHUNK, D), jnp.float32),
            pltpu.VMEM((16, D), jnp.float32),
            pltpu.VMEM((16,), jnp.int32),
            pltpu.SemaphoreType.DMA,
            pltpu.SemaphoreType.DMA,
            pltpu.SemaphoreType.DMA,
            pltpu.SemaphoreType.DMA,
            pltpu.SemaphoreType.DMA,
        ],
    )
    return run(cache2, latest2, pos)


def kernel(k_cache, v_cache, latest_k, latest_v, position_ids):
    pos = position_ids.reshape(B).astype(jnp.int32)
    v_new = _sc_update(
        v_cache.reshape(BH * S, D), latest_v.reshape(BH, D), pos)
    k_new = _tc_update(k_cache, latest_k, pos)
    return (k_new, v_new.reshape(B, H, S, D))
